# baseline (speedup 1.0000x reference)
import functools

import jax
import jax.numpy as jnp
from jax import lax
from jax.experimental import pallas as pl
from jax.experimental.pallas import tpu as pltpu
from jax.experimental.pallas import tpu_sc as plsc

_NUM_EMB = 1000000
_D = 64
_M = 99999
_P = 2147483647
_A = (98765431, 12345701)
_B = (7654321, 2468101)

_NC, _NS, _L = 2, 16, 16
_NW = _NC * _NS
_NROW, _W = 16384, 50
_N = _NROW * _W
_C = 128
_NPW = _N // _NW
_BPW = _NROW // _C // _NW
_G = _BPW * _W


def _srl(x, n):
    return lax.shift_right_logical(x, jnp.int32(n))


def _fold(v):
    return (v & _P) + _srl(v, 31)


def _hash16(x, a, b):
    a_hi, a_lo = a >> 16, a & 0xFFFF
    x_hi = _srl(x, 16)
    x_lo = x & 0xFFFF
    t0 = a_lo * x_lo
    t0m = _fold(t0)
    t0r = jnp.where(t0m >= _P, t0m - _P, t0m)
    t1 = a_hi * x_lo + a_lo * x_hi
    t1c = ((t1 & 0x7FFF) << 16) + _srl(t1, 15)
    t2 = (a_hi * x_hi) * 2
    s1 = t0r + t1c
    s1m = _fold(s1)
    s1r = jnp.where(s1m >= _P, s1m - _P, s1m)
    s2 = s1r + t2 + b
    s2m = _fold(s2)
    h = jnp.where(s2m >= _P, s2m - _P, s2m)
    return h % _M + 1


def _body(emb_hbm, pw_hbm, idx_hbm, out_hbm,
          idx_all, b0_v, b1_v, i0_v, i1_v, p0_v, p1_v, r0_v, r1_v, st_v,
          gsem, osem):
    wid = (lax.axis_index("s").astype(jnp.int32) * jnp.int32(_NC)
           + lax.axis_index("c").astype(jnp.int32))
    base_w = wid * jnp.int32(_NPW)

    pltpu.sync_copy(idx_hbm.at[pl.ds(base_w, _NPW)], idx_all)

    iota = lax.iota(jnp.int32, _L)
    iota_w = iota * jnp.int32(_W)

    def _grp(g):
        blk = g // jnp.int32(_W)
        i1 = g - blk * jnp.int32(_W)
        return blk, i1

    def _issue(g, s):
        blk, i1 = _grp(g)
        for kv in range(_C // _L):
            off = (blk * jnp.int32(_C) + jnp.int32(kv * _L)) * jnp.int32(_W) + i1
            x = plsc.load_gather(idx_all, [jnp.broadcast_to(off, (_L,)) + iota_w])
            b0_v[s][pl.ds(kv * _L, _L)] = _hash16(x, _A[0], _B[0])
            b1_v[s][pl.ds(kv * _L, _L)] = _hash16(x, _A[1], _B[1])
            i0_v[s][pl.ds(kv * _L, _L)] = x * 2
            i1_v[s][pl.ds(kv * _L, _L)] = x * 2 + 1
        pltpu.async_copy(pw_hbm.at[i0_v[s]], p0_v[s], gsem[s])
        pltpu.async_copy(pw_hbm.at[i1_v[s]], p1_v[s], gsem[s])
        pltpu.async_copy(emb_hbm.at[b0_v[s]], r0_v[s], gsem[s])
        pltpu.async_copy(emb_hbm.at[b1_v[s]], r1_v[s], gsem[s])

    def _drain_gathers(s):
        pltpu.make_async_copy(pw_hbm.at[i0_v[s]], p0_v[s], gsem[s]).wait()
        pltpu.make_async_copy(pw_hbm.at[i1_v[s]], p1_v[s], gsem[s]).wait()
        pltpu.make_async_copy(emb_hbm.at[b0_v[s]], r0_v[s], gsem[s]).wait()
        pltpu.make_async_copy(emb_hbm.at[b1_v[s]], r1_v[s], gsem[s]).wait()

    scat = [((jnp.int32(j * _L) + iota) // 8) * jnp.int32(1024)
            + ((jnp.int32(j * _L) + iota) % 8) * jnp.int32(_C)
            for j in range(_D // _L)]

    def _combine(s):
        @pl.loop(jnp.int32(0), jnp.int32(_C), step=jnp.int32(4))
        def _tok(t0):
            t0 = t0.astype(jnp.int32)
            for dt in range(4):
                t = t0 + jnp.int32(dt)
                tt = jnp.broadcast_to(t, (_L,))
                p0 = plsc.load_gather(p0_v[s], [tt])
                p1 = plsc.load_gather(p1_v[s], [tt])
                for j in range(_D // _L):
                    r0j = r0_v[s][t, pl.ds(j * _L, _L)]
                    r1j = r1_v[s][t, pl.ds(j * _L, _L)]
                    plsc.store_scatter(st_v[s], [scat[j] + tt],
                                       p0 * r0j + p1 * r1j)

    def _out_copies(g, s, sem):
        blk, i1 = _grp(g)
        i0b = wid * jnp.int32(_BPW) + blk
        return [pltpu.make_async_copy(
                    st_v[s].at[pl.ds(d8 * 1024, 1024)],
                    out_hbm.at[i1, jnp.int32(d8), i0b], sem)
                for d8 in range(8)]

    _issue(jnp.int32(0), 0)

    @pl.loop(jnp.int32(0), jnp.int32(_G), step=jnp.int32(2))
    def _ring(g0):
        g0 = g0.astype(jnp.int32)
        for s in range(2):
            g = g0 + jnp.int32(s)

            @pl.when(g + 1 < _G)
            def _():
                _issue(g + 1, 1 - s)

            _drain_gathers(s)

            @pl.when(g >= 2)
            def _():
                for c in _out_copies(g - 2, s, osem[s]):
                    c.wait()

            _combine(s)
            for c in _out_copies(g, s, osem[s]):
                c.start()

    for c in _out_copies(jnp.int32(_G - 2), 0, osem[0]):
        c.wait()
    for c in _out_copies(jnp.int32(_G - 1), 1, osem[1]):
        c.wait()


_mesh = plsc.VectorSubcoreMesh(
    core_axis_name="c", subcore_axis_name="s", num_cores=_NC, num_subcores=_NS)

_sc_call = pl.kernel(
    _body,
    out_type=jax.ShapeDtypeStruct((_W, 8, _NROW // _C, 1024), jnp.float32),
    mesh=_mesh,
    scratch_types=[
        pltpu.VMEM((_NPW,), jnp.int32),
        [pltpu.VMEM((_C,), jnp.int32) for _ in range(2)],
        [pltpu.VMEM((_C,), jnp.int32) for _ in range(2)],
        [pltpu.VMEM((_C,), jnp.int32) for _ in range(2)],
        [pltpu.VMEM((_C,), jnp.int32) for _ in range(2)],
        [pltpu.VMEM((_C,), jnp.float32) for _ in range(2)],
        [pltpu.VMEM((_C,), jnp.float32) for _ in range(2)],
        [pltpu.VMEM((_C, _D), jnp.float32) for _ in range(2)],
        [pltpu.VMEM((_C, _D), jnp.float32) for _ in range(2)],
        [pltpu.VMEM((8 * 1024,), jnp.float32) for _ in range(2)],
        [pltpu.SemaphoreType.DMA for _ in range(2)],
        [pltpu.SemaphoreType.DMA for _ in range(2)],
    ],
    compiler_params=pltpu.CompilerParams(
        needs_layout_passes=False, use_tc_tiling_on_sc=False),
)


def kernel(shared_embeddings, importance_weights, indices):
    idx32 = indices.reshape(-1).astype(jnp.int32)
    o = _sc_call(shared_embeddings.astype(jnp.float32),
                 importance_weights.astype(jnp.float32).reshape(-1), idx32)
    o = o.reshape(_W, 8, _NROW // _C, 8, _C)
    o = o.transpose(2, 4, 0, 1, 3)
    return o.reshape(_NROW, _W, _D)


# speedup vs baseline: 1.7660x; 1.7660x over previous
import functools

import jax
import jax.numpy as jnp
from jax import lax
from jax.experimental import pallas as pl
from jax.experimental.pallas import tpu as pltpu
from jax.experimental.pallas import tpu_sc as plsc

_NUM_EMB = 1000000
_D = 64
_M = 99999
_P = 2147483647
_A = (98765431, 12345701)
_B = (7654321, 2468101)

_NC, _NS, _L = 2, 16, 16
_NW = _NC * _NS
_NROW, _W = 16384, 50
_N = _NROW * _W
_C = 128
_NPW = _N // _NW
_BPW = _NROW // _C // _NW
_G = _BPW * _W


def _srl(x, n):
    return lax.shift_right_logical(x, jnp.int32(n))


def _fold(v):
    return (v & _P) + _srl(v, 31)


def _hash16(x, a, b):
    a_hi, a_lo = a >> 16, a & 0xFFFF
    x_hi = _srl(x, 16)
    x_lo = x & 0xFFFF
    t0 = a_lo * x_lo
    t0m = _fold(t0)
    t0r = jnp.where(t0m >= _P, t0m - _P, t0m)
    t1 = a_hi * x_lo + a_lo * x_hi
    t1c = ((t1 & 0x7FFF) << 16) + _srl(t1, 15)
    t2 = (a_hi * x_hi) * 2
    s1 = t0r + t1c
    s1m = _fold(s1)
    s1r = jnp.where(s1m >= _P, s1m - _P, s1m)
    s2 = s1r + t2 + b
    s2m = _fold(s2)
    h = jnp.where(s2m >= _P, s2m - _P, s2m)
    return h % _M + 1


def _body(emb_hbm, pw_hbm, idx_hbm, out_hbm,
          idx_all, b0_v, b1_v, i0_v, i1_v, p0_v, p1_v, r0_v, r1_v, st_v,
          gsem, osem):
    wid = (lax.axis_index("s").astype(jnp.int32) * jnp.int32(_NC)
           + lax.axis_index("c").astype(jnp.int32))
    base_w = wid * jnp.int32(_NPW)

    pltpu.sync_copy(idx_hbm.at[pl.ds(base_w, _NPW)], idx_all)

    iota = lax.iota(jnp.int32, _L)
    iota_w = iota * jnp.int32(_W)

    def _grp(g):
        blk = _srl(g * jnp.int32(1311), 16)
        i1 = g - blk * jnp.int32(_W)
        return blk, i1

    def _issue(g, s):
        blk, i1 = _grp(g)
        for kv in range(_C // _L):
            off = (blk * jnp.int32(_C) + jnp.int32(kv * _L)) * jnp.int32(_W) + i1
            x = plsc.load_gather(idx_all, [jnp.broadcast_to(off, (_L,)) + iota_w])
            b0_v[s][pl.ds(kv * _L, _L)] = _hash16(x, _A[0], _B[0])
            b1_v[s][pl.ds(kv * _L, _L)] = _hash16(x, _A[1], _B[1])
            i0_v[s][pl.ds(kv * _L, _L)] = x
            i1_v[s][pl.ds(kv * _L, _L)] = x + jnp.int32(_NUM_EMB)
        pltpu.async_copy(pw_hbm.at[i0_v[s]], p0_v[s], gsem[s])
        pltpu.async_copy(pw_hbm.at[i1_v[s]], p1_v[s], gsem[s])
        pltpu.async_copy(emb_hbm.at[b0_v[s]], r0_v[s], gsem[s])
        pltpu.async_copy(emb_hbm.at[b1_v[s]], r1_v[s], gsem[s])

    def _drain_gathers(s):
        pltpu.make_async_copy(pw_hbm.at[i0_v[s]], p0_v[s], gsem[s]).wait()
        pltpu.make_async_copy(pw_hbm.at[i1_v[s]], p1_v[s], gsem[s]).wait()
        pltpu.make_async_copy(emb_hbm.at[b0_v[s]], r0_v[s], gsem[s]).wait()
        pltpu.make_async_copy(emb_hbm.at[b1_v[s]], r1_v[s], gsem[s]).wait()

    dvecs = [jnp.int32(j * _L) + iota for j in range(_D // _L)]
    scat0 = [_srl(dv, 3) for dv in dvecs]
    scat1 = [(dv & 7) * jnp.int32(_C) for dv in dvecs]

    def _combine(s):
        @pl.loop(jnp.int32(0), jnp.int32(_C), step=jnp.int32(4))
        def _tok(t0):
            t0 = t0.astype(jnp.int32)
            for dt in range(4):
                t = t0 + jnp.int32(dt)
                tt = jnp.broadcast_to(t, (_L,))
                p0 = plsc.load_gather(p0_v[s], [tt])
                p1 = plsc.load_gather(p1_v[s], [tt])
                for j in range(_D // _L):
                    r0j = r0_v[s][t, pl.ds(j * _L, _L)]
                    r1j = r1_v[s][t, pl.ds(j * _L, _L)]
                    plsc.store_scatter(st_v[s], [scat0[j], scat1[j] + tt],
                                       p0 * r0j + p1 * r1j)

    def _out_copies(g, s, sem):
        blk, i1 = _grp(g)
        i0b = wid * jnp.int32(_BPW) + blk
        return [pltpu.make_async_copy(
                    st_v[s], out_hbm.at[i1, :, i0b], sem)]

    _issue(jnp.int32(0), 0)

    @pl.loop(jnp.int32(0), jnp.int32(_G), step=jnp.int32(2))
    def _ring(g0):
        g0 = g0.astype(jnp.int32)
        for s in range(2):
            g = g0 + jnp.int32(s)

            @pl.when(g + 1 < _G)
            def _():
                _issue(g + 1, 1 - s)

            _drain_gathers(s)

            @pl.when(g >= 2)
            def _():
                for c in _out_copies(g - 2, s, osem[s]):
                    c.wait()

            _combine(s)
            for c in _out_copies(g, s, osem[s]):
                c.start()

    for c in _out_copies(jnp.int32(_G - 2), 0, osem[0]):
        c.wait()
    for c in _out_copies(jnp.int32(_G - 1), 1, osem[1]):
        c.wait()


_mesh = plsc.VectorSubcoreMesh(
    core_axis_name="c", subcore_axis_name="s", num_cores=_NC, num_subcores=_NS)

_sc_call = pl.kernel(
    _body,
    out_type=jax.ShapeDtypeStruct((_W, 8, _NROW // _C, 1024), jnp.float32),
    mesh=_mesh,
    scratch_types=[
        pltpu.VMEM((_NPW,), jnp.int32),
        [pltpu.VMEM((_C,), jnp.int32) for _ in range(2)],
        [pltpu.VMEM((_C,), jnp.int32) for _ in range(2)],
        [pltpu.VMEM((_C,), jnp.int32) for _ in range(2)],
        [pltpu.VMEM((_C,), jnp.int32) for _ in range(2)],
        [pltpu.VMEM((_C,), jnp.float32) for _ in range(2)],
        [pltpu.VMEM((_C,), jnp.float32) for _ in range(2)],
        [pltpu.VMEM((_C, _D), jnp.float32) for _ in range(2)],
        [pltpu.VMEM((_C, _D), jnp.float32) for _ in range(2)],
        [pltpu.VMEM((8, 1024), jnp.float32) for _ in range(2)],
        [pltpu.SemaphoreType.DMA for _ in range(2)],
        [pltpu.SemaphoreType.DMA for _ in range(2)],
    ],
    compiler_params=pltpu.CompilerParams(
        needs_layout_passes=False, use_tc_tiling_on_sc=False),
)


def kernel(shared_embeddings, importance_weights, indices):
    idx32 = indices.reshape(-1).astype(jnp.int32)
    pw_flat = importance_weights.astype(jnp.float32).T.reshape(-1)
    o = _sc_call(shared_embeddings.astype(jnp.float32), pw_flat, idx32)
    o = o.reshape(_W, 8, _NROW // _C, 8, _C)
    o = o.transpose(2, 4, 0, 1, 3)
    return o.reshape(_NROW, _W, _D)


# speedup vs baseline: 2.4649x; 1.3958x over previous
import functools

import jax
import jax.numpy as jnp
from jax import lax
from jax.experimental import pallas as pl
from jax.experimental.pallas import tpu as pltpu
from jax.experimental.pallas import tpu_sc as plsc

_NUM_EMB = 1000000
_D = 64
_M = 99999
_P = 2147483647
_A = (98765431, 12345701)
_B = (7654321, 2468101)

_NC, _NS, _L = 2, 16, 16
_NW = _NC * _NS
_NROW, _W = 16384, 50
_N = _NROW * _W
_C = 128
_NPW = _N // _NW
_BPW = _NROW // _C // _NW
_G = _BPW * _W


def _srl(x, n):
    return lax.shift_right_logical(x, jnp.int32(n))


def _fold(v):
    return (v & _P) + _srl(v, 31)


def _hash16(x, a, b):
    a_hi, a_lo = a >> 16, a & 0xFFFF
    x_hi = _srl(x, 16)
    x_lo = x & 0xFFFF
    t0 = a_lo * x_lo
    t0m = _fold(t0)
    t0r = jnp.where(t0m >= _P, t0m - _P, t0m)
    t1 = a_hi * x_lo + a_lo * x_hi
    t1c = ((t1 & 0x7FFF) << 16) + _srl(t1, 15)
    t2 = (a_hi * x_hi) * 2
    s1 = t0r + t1c
    s1m = _fold(s1)
    s1r = jnp.where(s1m >= _P, s1m - _P, s1m)
    s2 = s1r + t2 + b
    s2m = _fold(s2)
    h = jnp.where(s2m >= _P, s2m - _P, s2m)
    return h % _M + 1


def _body(emb_hbm, pw_hbm, idx_hbm, out_hbm,
          idx_all, b0_v, b1_v, i0_v, i1_v, p0_v, p1_v, r0_v, r1_v, st_v,
          gsem, osem):
    wid = (lax.axis_index("s").astype(jnp.int32) * jnp.int32(_NC)
           + lax.axis_index("c").astype(jnp.int32))
    base_w = wid * jnp.int32(_NPW)

    pltpu.sync_copy(idx_hbm.at[pl.ds(base_w, _NPW)], idx_all)

    iota = lax.iota(jnp.int32, _L)
    iota_w = iota * jnp.int32(_W)

    def _grp(g):
        blk = _srl(g * jnp.int32(1311), 16)
        i1 = g - blk * jnp.int32(_W)
        return blk, i1

    def _issue(g, s):
        blk, i1 = _grp(g)
        for kv in range(_C // _L):
            off = (blk * jnp.int32(_C) + jnp.int32(kv * _L)) * jnp.int32(_W) + i1
            x = plsc.load_gather(idx_all, [jnp.broadcast_to(off, (_L,)) + iota_w])
            b0_v[s][pl.ds(kv * _L, _L)] = _hash16(x, _A[0], _B[0])
            b1_v[s][pl.ds(kv * _L, _L)] = _hash16(x, _A[1], _B[1])
            i0_v[s][pl.ds(kv * _L, _L)] = x
            i1_v[s][pl.ds(kv * _L, _L)] = x + jnp.int32(_NUM_EMB)
        pltpu.async_copy(pw_hbm.at[i0_v[s]], p0_v[s], gsem[s])
        pltpu.async_copy(pw_hbm.at[i1_v[s]], p1_v[s], gsem[s])
        pltpu.async_copy(emb_hbm.at[b0_v[s]], r0_v[s], gsem[s])
        pltpu.async_copy(emb_hbm.at[b1_v[s]], r1_v[s], gsem[s])

    def _drain_gathers(s):
        pltpu.make_async_copy(pw_hbm.at[i0_v[s]], p0_v[s], gsem[s]).wait()
        pltpu.make_async_copy(pw_hbm.at[i1_v[s]], p1_v[s], gsem[s]).wait()
        pltpu.make_async_copy(emb_hbm.at[b0_v[s]], r0_v[s], gsem[s]).wait()
        pltpu.make_async_copy(emb_hbm.at[b1_v[s]], r1_v[s], gsem[s]).wait()

    dvecs = [jnp.int32(j * _L) + iota for j in range(_D // _L)]
    scat0 = [_srl(dv, 3) for dv in dvecs]
    scat1 = [(dv & 7) * jnp.int32(_C) for dv in dvecs]

    def _combine(s):
        @plsc.parallel_loop(jnp.int32(0), jnp.int32(_C), jnp.int32(1), unroll=4)
        def _tok(t):
            t = t.astype(jnp.int32)
            tt = jnp.broadcast_to(t, (_L,))
            p0 = plsc.load_gather(p0_v[s], [tt])
            p1 = plsc.load_gather(p1_v[s], [tt])
            for j in range(_D // _L):
                r0j = r0_v[s][t, pl.ds(j * _L, _L)]
                r1j = r1_v[s][t, pl.ds(j * _L, _L)]
                plsc.store_scatter(st_v[s], [scat0[j], scat1[j] + tt],
                                   p0 * r0j + p1 * r1j)

    def _out_copies(g, s, sem):
        blk, i1 = _grp(g)
        i0b = wid * jnp.int32(_BPW) + blk
        return [pltpu.make_async_copy(
                    st_v[s], out_hbm.at[i1, :, i0b], sem)]

    _issue(jnp.int32(0), 0)

    @pl.loop(jnp.int32(0), jnp.int32(_G), step=jnp.int32(2))
    def _ring(g0):
        g0 = g0.astype(jnp.int32)
        for s in range(2):
            g = g0 + jnp.int32(s)

            @pl.when(g + 1 < _G)
            def _():
                _issue(g + 1, 1 - s)

            _drain_gathers(s)

            @pl.when(g >= 2)
            def _():
                for c in _out_copies(g - 2, s, osem[s]):
                    c.wait()

            _combine(s)
            for c in _out_copies(g, s, osem[s]):
                c.start()

    for c in _out_copies(jnp.int32(_G - 2), 0, osem[0]):
        c.wait()
    for c in _out_copies(jnp.int32(_G - 1), 1, osem[1]):
        c.wait()


_mesh = plsc.VectorSubcoreMesh(
    core_axis_name="c", subcore_axis_name="s", num_cores=_NC, num_subcores=_NS)

_sc_call = pl.kernel(
    _body,
    out_type=jax.ShapeDtypeStruct((_W, 8, _NROW // _C, 1024), jnp.float32),
    mesh=_mesh,
    scratch_types=[
        pltpu.VMEM((_NPW,), jnp.int32),
        [pltpu.VMEM((_C,), jnp.int32) for _ in range(2)],
        [pltpu.VMEM((_C,), jnp.int32) for _ in range(2)],
        [pltpu.VMEM((_C,), jnp.int32) for _ in range(2)],
        [pltpu.VMEM((_C,), jnp.int32) for _ in range(2)],
        [pltpu.VMEM((_C,), jnp.float32) for _ in range(2)],
        [pltpu.VMEM((_C,), jnp.float32) for _ in range(2)],
        [pltpu.VMEM((_C, _D), jnp.float32) for _ in range(2)],
        [pltpu.VMEM((_C, _D), jnp.float32) for _ in range(2)],
        [pltpu.VMEM((8, 1024), jnp.float32) for _ in range(2)],
        [pltpu.SemaphoreType.DMA for _ in range(2)],
        [pltpu.SemaphoreType.DMA for _ in range(2)],
    ],
    compiler_params=pltpu.CompilerParams(
        needs_layout_passes=False, use_tc_tiling_on_sc=False),
)


def kernel(shared_embeddings, importance_weights, indices):
    idx32 = indices.reshape(-1).astype(jnp.int32)
    pw_flat = importance_weights.astype(jnp.float32).T.reshape(-1)
    o = _sc_call(shared_embeddings.astype(jnp.float32), pw_flat, idx32)
    o = o.reshape(_W, 8, _NROW // _C, 8, _C)
    o = o.transpose(2, 4, 0, 1, 3)
    return o.reshape(_NROW, _W, _D)


# speedup vs baseline: 2.6978x; 1.0945x over previous
import functools

import jax
import jax.numpy as jnp
from jax import lax
from jax.experimental import pallas as pl
from jax.experimental.pallas import tpu as pltpu
from jax.experimental.pallas import tpu_sc as plsc

_NUM_EMB = 1000000
_D = 64
_M = 99999
_P = 2147483647
_A = (98765431, 12345701)
_B = (7654321, 2468101)

_NC, _NS, _L = 2, 16, 16
_NW = _NC * _NS
_NROW, _W = 16384, 50
_N = _NROW * _W
_C = 128
_NPW = _N // _NW
_BPW = _NROW // _C // _NW
_G = _BPW * _W


def _srl(x, n):
    return lax.shift_right_logical(x, jnp.int32(n))


def _fold(v):
    return (v & _P) + _srl(v, 31)


def _hash16(x, a, b):
    a_hi, a_lo = a >> 16, a & 0xFFFF
    x_hi = _srl(x, 16)
    x_lo = x & 0xFFFF
    t0 = a_lo * x_lo
    t0m = _fold(t0)
    t0r = jnp.where(t0m >= _P, t0m - _P, t0m)
    t1 = a_hi * x_lo + a_lo * x_hi
    t1c = ((t1 & 0x7FFF) << 16) + _srl(t1, 15)
    t2 = (a_hi * x_hi) * 2
    s1 = t0r + t1c
    s1m = _fold(s1)
    s1r = jnp.where(s1m >= _P, s1m - _P, s1m)
    s2 = s1r + t2 + b
    s2m = _fold(s2)
    h = jnp.where(s2m >= _P, s2m - _P, s2m)
    return h % _M + 1


def _body(emb_hbm, pw_hbm, idx_hbm, out_hbm,
          idx_all, b0_v, b1_v, i0_v, i1_v, p0_v, p1_v, r0_v, r1_v, st_v,
          gsem, osem):
    wid = (lax.axis_index("s").astype(jnp.int32) * jnp.int32(_NC)
           + lax.axis_index("c").astype(jnp.int32))
    base_w = wid * jnp.int32(_NPW)

    pltpu.sync_copy(idx_hbm.at[pl.ds(base_w, _NPW)], idx_all)

    iota = lax.iota(jnp.int32, _L)
    iota_w = iota * jnp.int32(_W)

    def _grp(g):
        blk = _srl(g * jnp.int32(1311), 16)
        i1 = g - blk * jnp.int32(_W)
        return blk, i1

    def _issue(g, s):
        blk, i1 = _grp(g)

        @plsc.parallel_loop(jnp.int32(0), jnp.int32(_C // _L), jnp.int32(1),
                            unroll=2)
        def _hash_kv(kv):
            kv = kv.astype(jnp.int32)
            off = (blk * jnp.int32(_C) + kv * jnp.int32(_L)) * jnp.int32(_W) + i1
            x = plsc.load_gather(idx_all, [jnp.broadcast_to(off, (_L,)) + iota_w])
            b0_v[s][pl.ds(kv * jnp.int32(_L), _L)] = _hash16(x, _A[0], _B[0])
            b1_v[s][pl.ds(kv * jnp.int32(_L), _L)] = _hash16(x, _A[1], _B[1])
            i0_v[s][pl.ds(kv * jnp.int32(_L), _L)] = x
            i1_v[s][pl.ds(kv * jnp.int32(_L), _L)] = x + jnp.int32(_NUM_EMB)
        pltpu.async_copy(pw_hbm.at[i0_v[s]], p0_v[s], gsem[s])
        pltpu.async_copy(pw_hbm.at[i1_v[s]], p1_v[s], gsem[s])
        pltpu.async_copy(emb_hbm.at[b0_v[s]], r0_v[s], gsem[s])
        pltpu.async_copy(emb_hbm.at[b1_v[s]], r1_v[s], gsem[s])

    def _drain_gathers(s):
        pltpu.make_async_copy(pw_hbm.at[i0_v[s]], p0_v[s], gsem[s]).wait()
        pltpu.make_async_copy(pw_hbm.at[i1_v[s]], p1_v[s], gsem[s]).wait()
        pltpu.make_async_copy(emb_hbm.at[b0_v[s]], r0_v[s], gsem[s]).wait()
        pltpu.make_async_copy(emb_hbm.at[b1_v[s]], r1_v[s], gsem[s]).wait()

    dvecs = [jnp.int32(j * _L) + iota for j in range(_D // _L)]
    scat0 = [_srl(dv, 3) for dv in dvecs]
    scat1 = [(dv & 7) * jnp.int32(_C) for dv in dvecs]

    def _combine(s, p):
        @plsc.parallel_loop(jnp.int32(0), jnp.int32(_C), jnp.int32(1), unroll=4)
        def _tok(t):
            t = t.astype(jnp.int32)
            tt = jnp.broadcast_to(t, (_L,))
            p0 = plsc.load_gather(p0_v[s], [tt])
            p1 = plsc.load_gather(p1_v[s], [tt])
            for j in range(_D // _L):
                r0j = r0_v[s][t, pl.ds(j * _L, _L)]
                r1j = r1_v[s][t, pl.ds(j * _L, _L)]
                plsc.store_scatter(st_v[p], [scat0[j], scat1[j] + tt],
                                   p0 * r0j + p1 * r1j)

    def _out_copies(g, s, sem):
        blk, i1 = _grp(g)
        i0b = wid * jnp.int32(_BPW) + blk
        return [pltpu.make_async_copy(
                    st_v[s], out_hbm.at[i1, :, i0b], sem)]

    _issue(jnp.int32(0), 0)
    _issue(jnp.int32(1), 1)

    @pl.loop(jnp.int32(0), jnp.int32(_G), step=jnp.int32(4))
    def _ring(g0):
        g0 = g0.astype(jnp.int32)
        for s in range(4):
            g = g0 + jnp.int32(s)
            p = s % 2

            @pl.when(g + 2 < _G)
            def _():
                _issue(g + 2, (s + 2) % 4)

            _drain_gathers(s)

            @pl.when(g >= 2)
            def _():
                for c in _out_copies(g - 2, p, osem[p]):
                    c.wait()

            _combine(s, p)
            for c in _out_copies(g, p, osem[p]):
                c.start()

    for c in _out_copies(jnp.int32(_G - 2), 0, osem[0]):
        c.wait()
    for c in _out_copies(jnp.int32(_G - 1), 1, osem[1]):
        c.wait()


_mesh = plsc.VectorSubcoreMesh(
    core_axis_name="c", subcore_axis_name="s", num_cores=_NC, num_subcores=_NS)

_sc_call = pl.kernel(
    _body,
    out_type=jax.ShapeDtypeStruct((_W, 8, _NROW // _C, 1024), jnp.float32),
    mesh=_mesh,
    scratch_types=[
        pltpu.VMEM((_NPW,), jnp.int32),
        [pltpu.VMEM((_C,), jnp.int32) for _ in range(4)],
        [pltpu.VMEM((_C,), jnp.int32) for _ in range(4)],
        [pltpu.VMEM((_C,), jnp.int32) for _ in range(4)],
        [pltpu.VMEM((_C,), jnp.int32) for _ in range(4)],
        [pltpu.VMEM((_C,), jnp.float32) for _ in range(4)],
        [pltpu.VMEM((_C,), jnp.float32) for _ in range(4)],
        [pltpu.VMEM((_C, _D), jnp.float32) for _ in range(4)],
        [pltpu.VMEM((_C, _D), jnp.float32) for _ in range(4)],
        [pltpu.VMEM((8, 1024), jnp.float32) for _ in range(2)],
        [pltpu.SemaphoreType.DMA for _ in range(4)],
        [pltpu.SemaphoreType.DMA for _ in range(2)],
    ],
    compiler_params=pltpu.CompilerParams(
        needs_layout_passes=False, use_tc_tiling_on_sc=False),
)


def kernel(shared_embeddings, importance_weights, indices):
    idx32 = indices.reshape(-1).astype(jnp.int32)
    pw_flat = importance_weights.astype(jnp.float32).T.reshape(-1)
    o = _sc_call(shared_embeddings.astype(jnp.float32), pw_flat, idx32)
    o = o.reshape(_W, 8, _NROW // _C, 8, _C)
    o = o.transpose(2, 4, 0, 1, 3)
    return o.reshape(_NROW, _W, _D)
